# baseline (speedup 1.0000x reference)
import functools

import jax
import jax.numpy as jnp
from jax import lax
from jax.experimental import pallas as pl
from jax.experimental.pallas import tpu as pltpu
from jax.experimental.pallas import tpu_sc as plsc

VOCAB = 1000000
DIM = 64
BATCH = 16384

NC = 2
NS = 16
L = 16
NW = NC * NS
BPW = BATCH // NW
CH = 128
HALF = BPW // 2


def _glove_body(i_hbm, j_hbm, ui_hbm, uj_hbm, bi_hbm, bj_hbm, out_hbm,
                idx_i, idx_j, q_i, q_j, rows_i, rows_j, b_i, b_j, out_v, sem):
    cid = lax.axis_index("c")
    sid = lax.axis_index("s")
    wid = sid * NC + cid
    base = wid * BPW

    pltpu.sync_copy(i_hbm.at[pl.ds(base, BPW)], idx_i)
    pltpu.sync_copy(j_hbm.at[pl.ds(base, BPW)], idx_j)

    def shift(t, carry):
        s = pl.ds(t * L, L)
        q_i[s] = lax.shift_right_logical(idx_i[s], 1)
        q_j[s] = lax.shift_right_logical(idx_j[s], 1)
        return carry

    lax.fori_loop(0, BPW // L, shift, 0)

    bias_copies = []
    for c in range(BPW // CH):
        s = pl.ds(c * CH, CH)
        bias_copies.append(pltpu.async_copy(bi_hbm.at[idx_i.at[s]], b_i.at[s], sem))
        bias_copies.append(pltpu.async_copy(bj_hbm.at[idx_j.at[s]], b_j.at[s], sem))

    for cp in bias_copies:
        cp.wait()

    lanes = lax.iota(jnp.int32, L)

    for p in range(2):
        copies = []
        for c in range(HALF // CH):
            src = pl.ds(p * HALF + c * CH, CH)
            dst = pl.ds(c * CH, CH)
            copies.append(pltpu.async_copy(ui_hbm.at[q_i.at[src]], rows_i.at[dst], sem))
            copies.append(pltpu.async_copy(uj_hbm.at[q_j.at[src]], rows_j.at[dst], sem))
        for cp in copies:
            cp.wait()

        def group(g, carry):
            s = pl.ds(p * HALF + g * L, L)
            vi = idx_i[s]
            vj = idx_j[s]
            ci = jnp.left_shift(jnp.bitwise_and(vi, 1), 6)
            cj = jnp.left_shift(jnp.bitwise_and(vj, 1), 6)
            rvec = g * L + lanes
            acc = b_i[s] + b_j[s]
            for d in range(DIM):
                a = plsc.load_gather(rows_i, [rvec, ci + d])
                b = plsc.load_gather(rows_j, [rvec, cj + d])
                acc = acc + a * b
            out_v[s] = acc
            return carry

        lax.fori_loop(0, HALF // L, group, 0)

    pltpu.sync_copy(out_v, out_hbm.at[pl.ds(base, BPW)])


_glove_call = pl.kernel(
    _glove_body,
    out_type=jax.ShapeDtypeStruct((BATCH,), jnp.float32),
    mesh=plsc.VectorSubcoreMesh(
        core_axis_name="c", subcore_axis_name="s", num_cores=NC, num_subcores=NS
    ),
    compiler_params=pltpu.CompilerParams(needs_layout_passes=False),
    scratch_types=[
        pltpu.VMEM((BPW,), jnp.int32),
        pltpu.VMEM((BPW,), jnp.int32),
        pltpu.VMEM((BPW,), jnp.int32),
        pltpu.VMEM((BPW,), jnp.int32),
        pltpu.VMEM((HALF, 2 * DIM), jnp.float32),
        pltpu.VMEM((HALF, 2 * DIM), jnp.float32),
        pltpu.VMEM((BPW,), jnp.float32),
        pltpu.VMEM((BPW,), jnp.float32),
        pltpu.VMEM((BPW,), jnp.float32),
        pltpu.SemaphoreType.DMA,
    ],
)


@jax.jit
def kernel(i_vecs, j_vecs, ui, uj, bi, bj):
    return _glove_call(i_vecs, j_vecs,
                       ui.reshape(VOCAB // 2, 2 * DIM),
                       uj.reshape(VOCAB // 2, 2 * DIM),
                       bi.reshape(VOCAB), bj.reshape(VOCAB))


# speedup vs baseline: 3.7836x; 3.7836x over previous
import functools

import jax
import jax.numpy as jnp
from jax import lax
from jax.experimental import pallas as pl
from jax.experimental.pallas import tpu as pltpu
from jax.experimental.pallas import tpu_sc as plsc

VOCAB = 1000000
DIM = 64
BATCH = 16384

NC = 2
NS = 16
L = 16
NW = NC * NS
BPW = BATCH // NW
NSLOT = 4
OROWS = 8
CH = 128

_MESH = plsc.VectorSubcoreMesh(
    core_axis_name="c", subcore_axis_name="s", num_cores=NC, num_subcores=NS
)
_PARAMS = pltpu.CompilerParams(
    needs_layout_passes=False, use_tc_tiling_on_sc=True,
    disable_bounds_checks=True,
)


def _gather_body(si_hbm, sj_hbm, ui_hbm, uj_hbm, rows_hbm,
                 tmp_v, runs_v, slots, staging, out_sem,
                 sem0, sem1, sem2, sem3):
    cid = lax.axis_index("c")
    sid = lax.axis_index("s")
    wid = sid * NC + cid
    base = wid * BPW
    lanes = lax.iota(jnp.int32, L)
    lane0 = jnp.equal(lanes, 0)
    sems = (sem0, sem1, sem2, sem3)

    def splat_at(ref, e):
        return plsc.load_gather(ref, [jnp.full((L,), e, jnp.int32)])

    def run_side(idx_hbm, tab_hbm, side):
        obase = side * BATCH * DIM + base * DIM

        pltpu.sync_copy(idx_hbm.at[pl.ds(base, BPW)], tmp_v)

        def scan(e, carry):
            n, prev = carry
            cv = lax.shift_right_logical(splat_at(tmp_v, e), 7)
            c = cv[0]
            is_new = jnp.not_equal(c, prev)
            packed = jnp.left_shift(cv, 9) + e
            plsc.store_scatter(
                runs_v, [jnp.full((L,), n, jnp.int32)], packed,
                mask=jnp.logical_and(lane0, is_new),
            )
            return jnp.where(is_new, n + 1, n), c

        n_runs, _ = lax.fori_loop(0, BPW, scan, (jnp.int32(0), jnp.int32(-1)))
        plsc.store_scatter(
            runs_v, [jnp.full((L,), n_runs, jnp.int32)],
            jnp.full((L,), BPW, jnp.int32), mask=lane0,
        )

        def fire(q):
            c = lax.shift_right_logical(splat_at(runs_v, q)[0], 9)
            cbase = pl.multiple_of(jnp.left_shift(c, 7), 128)
            slot = jnp.bitwise_and(q, NSLOT - 1)
            for k in range(NSLOT):
                @pl.when(jnp.equal(slot, k))
                def _():
                    pltpu.async_copy(
                        tab_hbm.at[:, pl.ds(cbase, 128)],
                        slots.at[pl.ds(k * DIM, DIM), :], sems[k]
                    )

        for q in range(NSLOT - 1):
            @pl.when(q < n_runs)
            def _():
                fire(q)

        def extract(e, carry2, k):
            colv = jnp.bitwise_and(splat_at(tmp_v, e), 127)
            srow = jnp.bitwise_and(e, OROWS - 1)
            for g in range(DIM // L):
                dvec = g * L + lanes
                v = plsc.load_gather(slots, [k * DIM + dvec, colv])
                staging[pl.ds(srow * DIM + g * L, L)] = v

            @pl.when(jnp.equal(jnp.bitwise_and(e, 1), 1))
            def _():
                sstart = jnp.bitwise_and(e - 1, OROWS - 1) * DIM
                off = pl.multiple_of(obase + (e - 1) * DIM, 2 * DIM)
                pltpu.async_copy(
                    staging.at[pl.ds(sstart, 2 * DIM)],
                    rows_hbm.at[pl.ds(off, 2 * DIM)],
                    out_sem,
                )

                @pl.when(e >= OROWS + 1)
                def _():
                    pltpu.make_async_copy(
                        staging.at[pl.ds(0, 2 * DIM)],
                        rows_hbm.at[pl.ds(0, 2 * DIM)],
                        out_sem,
                    ).wait()

            return carry2

        def do_run(r, carry):
            @pl.when(r + (NSLOT - 1) < n_runs)
            def _():
                fire(r + (NSLOT - 1))

            slot = jnp.bitwise_and(r, NSLOT - 1)
            start = jnp.bitwise_and(splat_at(runs_v, r)[0], 511)
            end = jnp.bitwise_and(splat_at(runs_v, r + 1)[0], 511)
            end = jnp.where(jnp.equal(end, 0), jnp.int32(BPW), end)

            for k in range(NSLOT):
                @pl.when(jnp.equal(slot, k))
                def _():
                    pltpu.make_async_copy(
                        tab_hbm.at[:, pl.ds(0, 128)],
                        slots.at[pl.ds(k * DIM, DIM), :], sems[k]
                    ).wait()
                    lax.fori_loop(
                        start, end, functools.partial(extract, k=k), 0
                    )

            return carry

        lax.fori_loop(0, n_runs, do_run, 0)

        for _ in range(OROWS // 2):
            pltpu.make_async_copy(
                staging.at[pl.ds(0, 2 * DIM)],
                rows_hbm.at[pl.ds(0, 2 * DIM)],
                out_sem,
            ).wait()

    run_side(si_hbm, ui_hbm, 0)
    run_side(sj_hbm, uj_hbm, 1)


_gather_call = pl.kernel(
    _gather_body,
    out_type=jax.ShapeDtypeStruct((2 * BATCH * DIM,), jnp.float32),
    mesh=_MESH,
    compiler_params=_PARAMS,
    scratch_types=[
        pltpu.VMEM((BPW,), jnp.int32),
        pltpu.VMEM((BPW + 8,), jnp.int32),
        pltpu.VMEM((NSLOT * DIM, 128), jnp.float32),
        pltpu.VMEM((OROWS * DIM,), jnp.float32),
        pltpu.SemaphoreType.DMA,
        pltpu.SemaphoreType.DMA,
        pltpu.SemaphoreType.DMA,
        pltpu.SemaphoreType.DMA,
        pltpu.SemaphoreType.DMA,
    ],
)


def _reduce_body(rows_hbm, invi_hbm, invj_hbm, i_hbm, j_hbm, bi_hbm, bj_hbm,
                 out_hbm, inv_i, inv_j, q_i, q_j, rows_i, rows_j,
                 idx_i, idx_j, b_i, b_j, out_v, sem):
    cid = lax.axis_index("c")
    sid = lax.axis_index("s")
    wid = sid * NC + cid
    base = wid * BPW

    pltpu.sync_copy(invi_hbm.at[pl.ds(base, BPW)], inv_i)
    pltpu.sync_copy(invj_hbm.at[pl.ds(base, BPW)], inv_j)
    pltpu.sync_copy(i_hbm.at[pl.ds(base, BPW)], idx_i)
    pltpu.sync_copy(j_hbm.at[pl.ds(base, BPW)], idx_j)

    def shift(t, carry):
        s = pl.ds(t * L, L)
        q_i[s] = lax.shift_right_logical(inv_i[s], 1)
        q_j[s] = lax.shift_right_logical(inv_j[s], 1) + (BATCH // 2)
        return carry

    lax.fori_loop(0, BPW // L, shift, 0)

    bias_copies = []
    for c in range(BPW // CH):
        s = pl.ds(c * CH, CH)
        bias_copies.append(pltpu.async_copy(bi_hbm.at[idx_i.at[s]], b_i.at[s], sem))
        bias_copies.append(pltpu.async_copy(bj_hbm.at[idx_j.at[s]], b_j.at[s], sem))
    for cp in bias_copies:
        cp.wait()

    lanes = lax.iota(jnp.int32, L)
    HALF = BPW // 2

    for p in range(2):
        copies = []
        for c in range(HALF // CH):
            src = pl.ds(p * HALF + c * CH, CH)
            dst = pl.ds(c * CH, CH)
            copies.append(pltpu.async_copy(rows_hbm.at[q_i.at[src]], rows_i.at[dst], sem))
            copies.append(pltpu.async_copy(rows_hbm.at[q_j.at[src]], rows_j.at[dst], sem))
        for cp in copies:
            cp.wait()

        def group(g, carry):
            s = pl.ds(p * HALF + g * L, L)
            vi = inv_i[s]
            vj = inv_j[s]
            ci = jnp.left_shift(jnp.bitwise_and(vi, 1), 6)
            cj = jnp.left_shift(jnp.bitwise_and(vj, 1), 6)
            rvec = g * L + lanes
            acc = b_i[s] + b_j[s]
            for d in range(DIM):
                a = plsc.load_gather(rows_i, [rvec, ci + d])
                b = plsc.load_gather(rows_j, [rvec, cj + d])
                acc = acc + a * b
            out_v[s] = acc
            return carry

        lax.fori_loop(0, HALF // L, group, 0)

    pltpu.sync_copy(out_v, out_hbm.at[pl.ds(base, BPW)])


_reduce_call = pl.kernel(
    _reduce_body,
    out_type=jax.ShapeDtypeStruct((BATCH,), jnp.float32),
    mesh=_MESH,
    compiler_params=_PARAMS,
    scratch_types=[
        pltpu.VMEM((BPW,), jnp.int32),
        pltpu.VMEM((BPW,), jnp.int32),
        pltpu.VMEM((BPW,), jnp.int32),
        pltpu.VMEM((BPW,), jnp.int32),
        pltpu.VMEM((BPW // 2, 2 * DIM), jnp.float32),
        pltpu.VMEM((BPW // 2, 2 * DIM), jnp.float32),
        pltpu.VMEM((BPW,), jnp.int32),
        pltpu.VMEM((BPW,), jnp.int32),
        pltpu.VMEM((BPW,), jnp.float32),
        pltpu.VMEM((BPW,), jnp.float32),
        pltpu.VMEM((BPW,), jnp.float32),
        pltpu.SemaphoreType.DMA,
    ],
)


@jax.jit
def kernel(i_vecs, j_vecs, ui, uj, bi, bj):
    arange = jnp.arange(BATCH, dtype=jnp.int32)
    pi = jnp.argsort(i_vecs).astype(jnp.int32)
    si = jnp.take(i_vecs, pi)
    inv_i = jnp.zeros((BATCH,), jnp.int32).at[pi].set(arange)
    pj = jnp.argsort(j_vecs).astype(jnp.int32)
    sj = jnp.take(j_vecs, pj)
    inv_j = jnp.zeros((BATCH,), jnp.int32).at[pj].set(arange)
    rows = _gather_call(si, sj, ui.T, uj.T)
    rows2d = rows.reshape(BATCH, 2 * DIM)
    return _reduce_call(rows2d, inv_i, inv_j, i_vecs, j_vecs,
                        bi.reshape(VOCAB), bj.reshape(VOCAB))
